# baseline (speedup 1.0000x reference)
import functools

import jax
import jax.numpy as jnp
from jax.experimental import pallas as pl
from jax.experimental.pallas import tpu as pltpu

_N = 4096
_D = 64
_L = 128
_K = 256
_S = 1024.0
_BIG = _S * _S
_MARGIN = 1.0


def _triplet_block_kernel(bm, nb, e_blk_ref, e_all_ref, t_blk_ref, t_all_ref,
                          out_ref, a_ref, b_ref, hist_ref):
    i = pl.program_id(0)
    lanes = jax.lax.broadcasted_iota(jnp.int32, (1, _L), 1)

    @pl.when(i == 0)
    def _build_b():
        ef = e_all_ref[...]
        tj = t_all_ref[...]
        oh_all = (tj == lanes).astype(jnp.float32)
        b_ref[:, 0:_D] = ef.astype(jnp.bfloat16)
        b_ref[:, _D:_D + 1] = jnp.sum(ef * ef, axis=1, keepdims=True
                                      ).astype(jnp.bfloat16)
        b_ref[:, _D + 1:_L] = jnp.zeros((_N, _L - _D - 1), jnp.bfloat16)
        b_ref[:, _L:_K] = (oh_all * _S).astype(jnp.bfloat16)
        hist_ref[...] = jnp.sum(oh_all, axis=0, keepdims=True)

    e = e_blk_ref[...]
    ti = t_blk_ref[...]
    oh_blk = (ti == lanes).astype(jnp.float32)
    a_ref[:, 0:_D] = (e * -2.0).astype(jnp.bfloat16)
    a_ref[:, _D:_D + 1] = jnp.ones((bm, 1), jnp.bfloat16)
    a_ref[:, _D + 1:_L] = jnp.zeros((bm, _L - _D - 1), jnp.bfloat16)
    a_ref[:, _L:_K] = (oh_blk * _S).astype(jnp.bfloat16)

    c = jax.lax.dot_general(
        a_ref[...], b_ref[...], (((1,), (1,)), ((), ())),
        preferred_element_type=jnp.float32)

    sq_i = jnp.sum(e * e, axis=1)
    pos_v = jnp.max(c, axis=1) + sq_i - _BIG
    neg_v = jnp.min(c, axis=1) + sq_i

    count = jnp.sum(oh_blk * hist_ref[...], axis=1)
    t0 = t_all_ref[0, 0]
    d0 = c[:, 0] + sq_i - jnp.where(ti[:, 0] == t0, _BIG, 0.0)
    ap = jnp.where(count > 1.5, pos_v, d0)
    an = jnp.where(count < _N - 0.5, neg_v, d0)

    losses = jnp.maximum(ap - an + _MARGIN, 0.0)
    s = jnp.sum(losses)

    @pl.when(i == 0)
    def _init_out():
        out_ref[...] = jnp.zeros((1, 1, 1), jnp.float32)

    acc = out_ref[0, 0, 0] + s
    out_ref[...] = jnp.where(i == nb - 1, acc / _N, acc).reshape(1, 1, 1)


def _triplet_mean_loss(embeddings, target, bm):
    nb = _N // bm
    tcol = target.astype(jnp.int32).reshape(_N, 1)
    out = pl.pallas_call(
        functools.partial(_triplet_block_kernel, bm, nb),
        grid=(nb,),
        in_specs=[
            pl.BlockSpec((bm, _D), lambda i: (i, 0)),
            pl.BlockSpec((_N, _D), lambda i: (0, 0)),
            pl.BlockSpec((bm, 1), lambda i: (i, 0)),
            pl.BlockSpec((_N, 1), lambda i: (0, 0)),
        ],
        out_specs=pl.BlockSpec((1, 1, 1), lambda i: (0, 0, 0)),
        out_shape=jax.ShapeDtypeStruct((1, 1, 1), jnp.float32),
        scratch_shapes=[
            pltpu.VMEM((bm, _K), jnp.bfloat16),
            pltpu.VMEM((_N, _K), jnp.bfloat16),
            pltpu.VMEM((1, _L), jnp.float32),
        ],
    )(embeddings, embeddings, tcol, tcol)
    return out.reshape(())


def kernel(embeddings, target):
    mean_loss = _triplet_mean_loss(embeddings, target, bm=2048)
    return (mean_loss, _N)


# speedup vs baseline: 1.0386x; 1.0386x over previous
import functools

import jax
import jax.numpy as jnp
from jax.experimental import pallas as pl
from jax.experimental.pallas import tpu as pltpu

_N = 4096
_D = 64
_L = 128
_K = 256
_S = 1024.0
_BIG = _S * _S
_MARGIN = 1.0


def _triplet_block_kernel(bm, nb, e_blk_ref, e_all_ref, t_blk_ref, t_all_ref,
                          out_ref, a_ref, b_ref, hist_ref):
    i = pl.program_id(0)
    lanes = jax.lax.broadcasted_iota(jnp.int32, (1, _L), 1)

    @pl.when(i == 0)
    def _build_b():
        ef = e_all_ref[...]
        tj = t_all_ref[...]
        oh_all = (tj == lanes).astype(jnp.float32)
        b_ref[:, 0:_D] = ef.astype(jnp.bfloat16)
        b_ref[:, _D:_D + 1] = jnp.sum(ef * ef, axis=1, keepdims=True
                                      ).astype(jnp.bfloat16)
        b_ref[:, _D + 1:_L] = jnp.zeros((_N, _L - _D - 1), jnp.bfloat16)
        b_ref[:, _L:_K] = (oh_all * _S).astype(jnp.bfloat16)
        hist_ref[...] = jnp.sum(oh_all, axis=0, keepdims=True)

    e = e_blk_ref[...]
    ti = t_blk_ref[...]
    oh_blk = (ti == lanes).astype(jnp.float32)
    a_ref[:, 0:_D] = (e * -2.0).astype(jnp.bfloat16)
    a_ref[:, _D:_D + 1] = jnp.ones((bm, 1), jnp.bfloat16)
    a_ref[:, _D + 1:_L] = jnp.zeros((bm, _L - _D - 1), jnp.bfloat16)
    a_ref[:, _L:_K] = (oh_blk * _S).astype(jnp.bfloat16)

    c = jax.lax.dot_general(
        a_ref[...], b_ref[...], (((1,), (1,)), ((), ())),
        preferred_element_type=jnp.float32)

    sq_i = jnp.sum(e * e, axis=1)
    pos_v = jnp.max(c, axis=1) + sq_i - _BIG
    neg_v = jnp.min(c, axis=1) + sq_i

    count = jnp.sum(oh_blk * hist_ref[...], axis=1)
    t0 = t_all_ref[0, 0]
    d0 = c[:, 0] + sq_i - jnp.where(ti[:, 0] == t0, _BIG, 0.0)
    ap = jnp.where(count > 1.5, pos_v, d0)
    an = jnp.where(count < _N - 0.5, neg_v, d0)

    losses = jnp.maximum(ap - an + _MARGIN, 0.0)
    s = jnp.sum(losses)

    @pl.when(i == 0)
    def _init_out():
        out_ref[...] = jnp.zeros((1, 1, 1), jnp.float32)

    acc = out_ref[0, 0, 0] + s
    out_ref[...] = jnp.where(i == nb - 1, acc / _N, acc).reshape(1, 1, 1)


def _triplet_mean_loss(embeddings, target, bm):
    nb = _N // bm
    tcol = target.astype(jnp.int32).reshape(_N, 1)
    out = pl.pallas_call(
        functools.partial(_triplet_block_kernel, bm, nb),
        grid=(nb,),
        in_specs=[
            pl.BlockSpec((bm, _D), lambda i: (i, 0)),
            pl.BlockSpec((_N, _D), lambda i: (0, 0)),
            pl.BlockSpec((bm, 1), lambda i: (i, 0)),
            pl.BlockSpec((_N, 1), lambda i: (0, 0)),
        ],
        out_specs=pl.BlockSpec((1, 1, 1), lambda i: (0, 0, 0)),
        out_shape=jax.ShapeDtypeStruct((1, 1, 1), jnp.float32),
        scratch_shapes=[
            pltpu.VMEM((bm, _K), jnp.bfloat16),
            pltpu.VMEM((_N, _K), jnp.bfloat16),
            pltpu.VMEM((1, _L), jnp.float32),
        ],
    )(embeddings, embeddings, tcol, tcol)
    return out.reshape(())


def kernel(embeddings, target):
    mean_loss = _triplet_mean_loss(embeddings, target, bm=512)
    return (mean_loss, _N)
